# Initial kernel scaffold; baseline (speedup 1.0000x reference)
import jax
import jax.numpy as jnp
from jax.experimental import pallas as pl


def kernel(zf, zx, ze, Wf1, bf1, Wf2, bf2, Wx1, bx1, Wx2, bx2, We1, be1, We2, be2):
    raise NotImplementedError("write your pallas kernel here")


# speedup vs baseline: 241.5754x; 241.5754x over previous
import jax
import jax.numpy as jnp
from jax.experimental import pallas as pl
from jax.experimental.pallas import tpu as pltpu

_T = 2000
_K = 20


def _mlp_rows(z_ref, w1_ref, b1_ref, w2t_ref, b2_ref, b):
    z = z_ref[b, 0]
    h = jnp.tanh(
        jax.lax.dot_general(w1_ref[...], z, (((0,), (1,)), ((), ())),
                            preferred_element_type=jnp.float32)
        + b1_ref[...])
    return jnp.tanh(
        jax.lax.dot_general(w2t_ref[...], h, (((1,), (0,)), ((), ())),
                            preferred_element_type=jnp.float32)
        + b2_ref[...])


def _phase1(zx_ref, ze_ref, wx1_ref, bx1_ref, wx2t_ref, bx2_ref,
            we1_ref, be1_ref, we2t_ref, be2_ref,
            x_ref, e_ref, part_ref):
    esum = jnp.float32(0.0)
    esq = jnp.float32(0.0)
    nb = zx_ref.shape[0]
    for b in range(nb):
        xb = _mlp_rows(zx_ref, wx1_ref, bx1_ref, wx2t_ref, bx2_ref, b)
        x_ref[0, b:b + 1, :] = xb
        part_ref[0, 0, b] = jnp.min(xb)
        part_ref[0, 0, nb + b] = jnp.max(xb)
        eb = _mlp_rows(ze_ref, we1_ref, be1_ref, we2t_ref, be2_ref, b)
        e_ref[0, b:b + 1, :] = eb
        esum = esum + jnp.sum(eb)
        esq = esq + jnp.sum(eb * eb)
    part_ref[0, 0, 2 * nb] = esum
    part_ref[0, 0, 2 * nb + 1] = esq


def _phase2(x_ref, e_ref, a_ref, w_ref, s_ref, base_ref, alpha_ref, y_ref):
    x = x_ref[0]
    e = e_ref[0]
    y = base_ref[...] + alpha_ref[0, 0] * e
    for s in range(_K - 1):
        t = x - a_ref[:, s:s + 1]
        t = jnp.maximum(jnp.minimum(t, w_ref[:, s:s + 1]), jnp.float32(0.0))
        y = y + s_ref[:, s:s + 1] * t
    y_ref[0] = y


def kernel(zf, zx, ze, Wf1, bf1, Wf2, bf2, Wx1, bx1, Wx2, bx2,
           We1, be1, We2, be2):
    B, N, L = zx.shape
    T = _T
    C = N // T
    f32 = jnp.float32

    pts = jnp.tanh(jnp.tanh(zf @ Wf1 + bf1) @ Wf2 + bf2)
    K = pts.shape[1]
    dirs = jax.random.randint(jax.random.key(42), (B,), 0, 2).astype(bool)
    srt = jnp.sort(pts, axis=1)
    yp = jnp.where(dirs[:, None], srt, srt[:, ::-1])
    xp = jnp.linspace(0.0, 1.0, K).astype(f32)

    zx4 = zx.reshape(B, C, T, L)
    ze4 = ze.reshape(B, C, T, L)
    bx1c = bx1.reshape(L, 1).astype(f32)
    be1c = be1.reshape(L, 1).astype(f32)
    wx2t = Wx2.T
    we2t = We2.T
    bx2c = bx2.reshape(1, 1)
    be2c = be2.reshape(1, 1)

    chunk_spec = pl.BlockSpec((B, 1, T, L), lambda i: (0, i, 0, 0))
    full = lambda shp: pl.BlockSpec(shp, lambda i: (0,) * len(shp))
    buf_spec = pl.BlockSpec((1, B, T), lambda i: (i, 0, 0))

    xbuf, ebuf, parts = pl.pallas_call(
        _phase1,
        grid=(C,),
        in_specs=[
            chunk_spec, chunk_spec,
            full((L, L)), full((L, 1)), full((1, L)), full((1, 1)),
            full((L, L)), full((L, 1)), full((1, L)), full((1, 1)),
        ],
        out_specs=[
            buf_spec, buf_spec,
            pl.BlockSpec((1, 1, 16), lambda i: (i, 0, 0),
                         memory_space=pltpu.SMEM),
        ],
        out_shape=[
            jax.ShapeDtypeStruct((C, B, T), f32),
            jax.ShapeDtypeStruct((C, B, T), f32),
            jax.ShapeDtypeStruct((C, 1, 16), f32),
        ],
    )(zx4, ze4, Wx1, bx1c, wx2t, bx2c, We1, be1c, we2t, be2c)

    parts = parts[:, 0, :]
    mn = jnp.min(parts[:, 0:B], axis=0)
    mx = jnp.max(parts[:, B:2 * B], axis=0)
    S = jnp.sum(parts[:, 2 * B])
    S2 = jnp.sum(parts[:, 2 * B + 1])
    ntot = f32(B * N)
    mean = S / ntot
    var = (S2 - S * S / ntot) / (ntot - f32(1.0))
    std = jnp.sqrt(var)
    alpha = (f32(0.1) / std).reshape(1, 1)
    beta = -f32(0.1) * mean / std

    D = (mx - mn)[:, None]
    w = (xp[1:] - xp[:-1])[None, :]
    slope = (yp[:, 1:] - yp[:, :-1]) / (w + f32(1e-7))
    a_tbl = mn[:, None] + xp[None, :K - 1] * D
    w_tbl = w * D
    s_tbl = slope / D
    base = yp[:, 0:1] + beta

    ybuf = pl.pallas_call(
        _phase2,
        grid=(C,),
        in_specs=[
            buf_spec, buf_spec,
            full((B, K - 1)), full((B, K - 1)), full((B, K - 1)),
            full((B, 1)),
            pl.BlockSpec((1, 1), lambda i: (0, 0), memory_space=pltpu.SMEM),
        ],
        out_specs=buf_spec,
        out_shape=jax.ShapeDtypeStruct((C, B, T), f32),
    )(xbuf, ebuf, a_tbl, w_tbl, s_tbl, base, alpha)

    return ybuf.transpose(0, 2, 1).reshape(N, B)
